# baseline (speedup 1.0000x reference)
import functools

import jax
import jax.numpy as jnp
from jax import lax
from jax.experimental import pallas as pl
from jax.experimental.pallas import tpu as pltpu

N = 50000
E = 800000
EA = 16
H = 64
L = 32
C = 10
G = 128

E_ROWS = E // 128
R_PAD = 6272
E_PAD = R_PAD * 128
TRASH = N
N1_PAD = 50048
BN = 2000
BA = 3200


def _proj_body(ea_ref, w1_ref, b1_ref, w2_ref, b2_ref, e1_ref, e2_ref):
    ea = ea_ref[...]
    e1 = jnp.dot(ea, w1_ref[...], preferred_element_type=jnp.float32) + b1_ref[...]
    e1_ref[...] = e1
    e2 = jnp.dot(ea, w2_ref[...], preferred_element_type=jnp.float32) + b2_ref[...]
    e2_ref[0] = e2[:, :32]
    e2_ref[1] = e2[:, 32:]


def _edge_proj(edge_attr, ep1_W, ep1_b, ep2_W, ep2_b):
    grid = E // BA
    return pl.pallas_call(
        _proj_body,
        grid=(grid,),
        in_specs=[
            pl.BlockSpec((BA, EA), lambda i: (i, 0)),
            pl.BlockSpec((EA, 1), lambda i: (0, 0)),
            pl.BlockSpec((1, 1), lambda i: (0, 0)),
            pl.BlockSpec((EA, H), lambda i: (0, 0)),
            pl.BlockSpec((1, H), lambda i: (0, 0)),
        ],
        out_specs=[
            pl.BlockSpec((BA, 1), lambda i: (i, 0)),
            pl.BlockSpec((2, BA, 32), lambda i: (0, i, 0)),
        ],
        out_shape=[
            jax.ShapeDtypeStruct((E_PAD, 1), jnp.float32),
            jax.ShapeDtypeStruct((2, E_PAD, 32), jnp.float32),
        ],
    )(edge_attr, ep1_W, ep1_b.reshape(1, 1), ep2_W, ep2_b.reshape(1, H))


def _mlp1_body(x_ref, a_ref, eps_ref, w1_ref, b1_ref, w2_ref, b2_ref, o_ref):
    g = (1.0 + eps_ref[0, 0]) * x_ref[...] + a_ref[0] + a_ref[1]
    h = jax.nn.relu(g * w1_ref[...] + b1_ref[...])
    h = jnp.dot(h, w2_ref[...], preferred_element_type=jnp.float32) + b2_ref[...]
    h = jnp.where(h > 0, h, 0.1 * h)
    o_ref[0] = h[:, :32]
    o_ref[1] = h[:, 32:]


def _mlp1(x2, agg1p, eps1, m1_W1, m1_b1, m1_W2, m1_b2):
    grid = N // BN
    return pl.pallas_call(
        _mlp1_body,
        grid=(grid,),
        in_specs=[
            pl.BlockSpec((BN, 1), lambda i: (i, 0)),
            pl.BlockSpec((2, BN, 1), lambda i: (0, i, 0)),
            pl.BlockSpec((1, 1), lambda i: (0, 0)),
            pl.BlockSpec((1, H), lambda i: (0, 0)),
            pl.BlockSpec((1, H), lambda i: (0, 0)),
            pl.BlockSpec((H, H), lambda i: (0, 0)),
            pl.BlockSpec((1, H), lambda i: (0, 0)),
        ],
        out_specs=pl.BlockSpec((2, BN, 32), lambda i: (0, i, 0)),
        out_shape=jax.ShapeDtypeStruct((2, N, 32), jnp.float32),
    )(x2, agg1p, eps1.reshape(1, 1), m1_W1, m1_b1.reshape(1, H), m1_W2,
      m1_b2.reshape(1, H))


def _mlp2_body(h_ref, a_ref, eps_ref, w1_ref, b1_ref, w2_ref, b2_ref,
               muW_ref, mub_ref, lvW_ref, lvb_ref, nz_ref,
               z_ref, mu_ref, lv_ref):
    h1 = jnp.concatenate([h_ref[0], h_ref[1]], axis=1)
    agg = jnp.concatenate([a_ref[0], a_ref[1]], axis=1)
    h = (1.0 + eps_ref[0, 0]) * h1 + agg
    t = jax.nn.relu(jnp.dot(h, w1_ref[...], preferred_element_type=jnp.float32)
                    + b1_ref[...])
    h2 = jnp.dot(t, w2_ref[...], preferred_element_type=jnp.float32) + b2_ref[...]
    h2 = jnp.where(h2 > 0, h2, 0.1 * h2)
    mu = jnp.dot(h2, muW_ref[...], preferred_element_type=jnp.float32) + mub_ref[...]
    lv = jnp.dot(h2, lvW_ref[...], preferred_element_type=jnp.float32) + lvb_ref[...]
    std = jnp.exp(0.5 * lv)
    z_ref[...] = mu + nz_ref[...] * std
    mu_ref[...] = mu
    lv_ref[...] = lv


def _mlp2_heads(h1s, agg2, eps2, m2_W1, m2_b1, m2_W2, m2_b2,
                mu_W, mu_b, lv_W, lv_b, noise_eps):
    grid = N // BN
    wspec = lambda r, c: pl.BlockSpec((r, c), lambda i: (0, 0))
    return pl.pallas_call(
        _mlp2_body,
        grid=(grid,),
        in_specs=[
            pl.BlockSpec((2, BN, 32), lambda i: (0, i, 0)),
            pl.BlockSpec((2, BN, 32), lambda i: (0, i, 0)),
            wspec(1, 1), wspec(H, H), wspec(1, H), wspec(H, H), wspec(1, H),
            wspec(H, L), wspec(1, L), wspec(H, L), wspec(1, L),
            pl.BlockSpec((BN, L), lambda i: (i, 0)),
        ],
        out_specs=[pl.BlockSpec((BN, L), lambda i: (i, 0))] * 3,
        out_shape=[jax.ShapeDtypeStruct((N, L), jnp.float32)] * 3,
    )(h1s, agg2, eps2.reshape(1, 1), m2_W1, m2_b1.reshape(1, H), m2_W2,
      m2_b2.reshape(1, H), mu_W, mu_b.reshape(1, L), lv_W, lv_b.reshape(1, L),
      noise_eps)


def _pool_body(z_ref, b_ref, cw_ref, cb_ref, o_ref, sums, counts):
    i = pl.program_id(0)

    @pl.when(i == 0)
    def _():
        sums[...] = jnp.zeros_like(sums)
        counts[...] = jnp.zeros_like(counts)

    iota = lax.broadcasted_iota(jnp.int32, (BN, G), 1)
    onehot = (iota == b_ref[...]).astype(jnp.float32)
    dn = (((0,), (0,)), ((), ()))
    zb = z_ref[...]
    zb = jnp.clip(jnp.where(jnp.isnan(zb), 0.0, zb), -1e38, 1e38)
    sums[...] += lax.dot_general(onehot, zb, dn,
                                 preferred_element_type=jnp.float32)
    counts[...] += lax.dot_general(onehot, jnp.ones((BN, 1), jnp.float32), dn,
                                   preferred_element_type=jnp.float32)

    @pl.when(i == pl.num_programs(0) - 1)
    def _():
        ge = sums[...] / jnp.maximum(counts[...], 1.0)
        o_ref[...] = jnp.dot(ge, cw_ref[...],
                             preferred_element_type=jnp.float32) + cb_ref[...]


def _pool_cls(z, batch2, cls_W, cls_b):
    grid = N // BN
    return pl.pallas_call(
        _pool_body,
        grid=(grid,),
        in_specs=[
            pl.BlockSpec((BN, L), lambda i: (i, 0)),
            pl.BlockSpec((BN, 1), lambda i: (i, 0)),
            pl.BlockSpec((L, C), lambda i: (0, 0)),
            pl.BlockSpec((1, C), lambda i: (0, 0)),
        ],
        out_specs=pl.BlockSpec((G, C), lambda i: (0, 0)),
        out_shape=jax.ShapeDtypeStruct((G, C), jnp.float32),
        scratch_shapes=[
            pltpu.VMEM((G, L), jnp.float32),
            pltpu.VMEM((G, 1), jnp.float32),
        ],
    )(z, batch2, cls_W, cls_b.reshape(1, C))


def kernel(x, edge_index, edge_attr, batch,
           eps1, ep1_W, ep1_b, m1_W1, m1_b1, m1_W2, m1_b2,
           eps2, ep2_W, ep2_b, m2_W1, m2_b1, m2_W2, m2_b2,
           mu_W, mu_b, lv_W, lv_b, cls_W, cls_b, noise_eps):
    src = edge_index[0].astype(jnp.int32)
    dst = edge_index[1].astype(jnp.int32)

    e1, e2 = _edge_proj(edge_attr, ep1_W, ep1_b, ep2_W, ep2_b)

    e1f = e1[:E, 0]
    msg1 = jax.nn.relu(jnp.take(x, src, axis=0) + e1f)
    agg1 = jax.ops.segment_sum(msg1, dst, num_segments=N)
    agg1p = jnp.stack([agg1, jnp.zeros_like(agg1)]).reshape(2, N, 1)

    h1s = _mlp1(x.reshape(N, 1), agg1p, eps1, m1_W1, m1_b1, m1_W2, m1_b2)

    h_flat = h1s.reshape(2 * N, 32)
    e2e = jnp.concatenate([e2[0, :E], e2[1, :E]], axis=1)
    h1full = jnp.concatenate([h1s[0], h1s[1]], axis=1)
    msg2 = jax.nn.relu(jnp.take(h1full, src, axis=0) + e2e)
    agg2f = jax.ops.segment_sum(msg2, dst, num_segments=N)
    agg2 = jnp.stack([agg2f[:, :32], agg2f[:, 32:]])

    z, mu, lv = _mlp2_heads(h1s, agg2, eps2, m2_W1, m2_b1, m2_W2, m2_b2,
                            mu_W, mu_b, lv_W, lv_b, noise_eps)

    logits = _pool_cls(z, batch.reshape(N, 1), cls_W, cls_b)
    return (z, mu, lv, logits)


# speedup vs baseline: 2.0560x; 2.0560x over previous
import functools

import jax
import jax.numpy as jnp
from jax import lax
from jax.experimental import pallas as pl
from jax.experimental.pallas import tpu as pltpu
from jax.experimental.pallas import tpu_sc as plsc

N = 50000
E = 800000
EA = 16
H = 64
L = 32
C = 10
G = 128

E_ROWS = E // 128
R_PAD = 6400
E_PAD = R_PAD * 128
TRASH = N
N1_PAD = 50176
BN = 2000
BA = 3200


def _proj_body(ea_ref, w1_ref, b1_ref, w2_ref, b2_ref, e1_ref, e2a_ref,
               e2b_ref):
    ea = ea_ref[...]
    e1 = jnp.dot(ea, w1_ref[...], preferred_element_type=jnp.float32) + b1_ref[...]
    e1_ref[...] = e1
    e2 = jnp.dot(ea, w2_ref[...], preferred_element_type=jnp.float32) + b2_ref[...]
    e2a_ref[...] = e2[:, :32]
    e2b_ref[...] = e2[:, 32:]


def _edge_proj(edge_attr, ep1_W, ep1_b, ep2_W, ep2_b):
    grid = E // BA
    return pl.pallas_call(
        _proj_body,
        grid=(grid,),
        in_specs=[
            pl.BlockSpec((BA, EA), lambda i: (i, 0)),
            pl.BlockSpec((EA, 1), lambda i: (0, 0)),
            pl.BlockSpec((1, 1), lambda i: (0, 0)),
            pl.BlockSpec((EA, H), lambda i: (0, 0)),
            pl.BlockSpec((1, H), lambda i: (0, 0)),
        ],
        out_specs=[
            pl.BlockSpec((BA, 1), lambda i: (i, 0)),
            pl.BlockSpec((BA, 32), lambda i: (i, 0)),
            pl.BlockSpec((BA, 32), lambda i: (i, 0)),
        ],
        out_shape=[
            jax.ShapeDtypeStruct((E_PAD, 1), jnp.float32),
            jax.ShapeDtypeStruct((E_PAD, 32), jnp.float32),
            jax.ShapeDtypeStruct((E_PAD, 32), jnp.float32),
        ],
    )(edge_attr, ep1_W, ep1_b.reshape(1, 1), ep2_W, ep2_b.reshape(1, H))


def _mlp1_body(x_ref, a_ref, eps_ref, w1_ref, b1_ref, w2_ref, b2_ref, o_ref):
    g = (1.0 + eps_ref[0, 0]) * x_ref[...] + a_ref[0] + a_ref[1]
    h = jax.nn.relu(g * w1_ref[...] + b1_ref[...])
    h = jnp.dot(h, w2_ref[...], preferred_element_type=jnp.float32) + b2_ref[...]
    h = jnp.where(h > 0, h, 0.1 * h)
    o_ref[...] = jnp.concatenate([h, jnp.zeros_like(h)], axis=1)


def _mlp1(x2, agg1p, eps1, m1_W1, m1_b1, m1_W2, m1_b2):
    grid = N // BN
    return pl.pallas_call(
        _mlp1_body,
        grid=(grid,),
        in_specs=[
            pl.BlockSpec((BN, 1), lambda i: (i, 0)),
            pl.BlockSpec((2, BN, 1), lambda i: (0, i, 0)),
            pl.BlockSpec((1, 1), lambda i: (0, 0)),
            pl.BlockSpec((1, H), lambda i: (0, 0)),
            pl.BlockSpec((1, H), lambda i: (0, 0)),
            pl.BlockSpec((H, H), lambda i: (0, 0)),
            pl.BlockSpec((1, H), lambda i: (0, 0)),
        ],
        out_specs=pl.BlockSpec((BN, 2 * H), lambda i: (i, 0)),
        out_shape=jax.ShapeDtypeStruct((N, 2 * H), jnp.float32),
    )(x2, agg1p, eps1.reshape(1, 1), m1_W1, m1_b1.reshape(1, H), m1_W2,
      m1_b2.reshape(1, H))


def _mlp2_body(h_ref, a_ref, eps_ref, w1_ref, b1_ref, w2_ref, b2_ref,
               muW_ref, mub_ref, lvW_ref, lvb_ref, nz_ref,
               z_ref, mu_ref, lv_ref):
    h1 = h_ref[:, :H]
    agg = jnp.concatenate([a_ref[0], a_ref[1]], axis=1)
    h = (1.0 + eps_ref[0, 0]) * h1 + agg
    t = jax.nn.relu(jnp.dot(h, w1_ref[...], preferred_element_type=jnp.float32)
                    + b1_ref[...])
    h2 = jnp.dot(t, w2_ref[...], preferred_element_type=jnp.float32) + b2_ref[...]
    h2 = jnp.where(h2 > 0, h2, 0.1 * h2)
    mu = jnp.dot(h2, muW_ref[...], preferred_element_type=jnp.float32) + mub_ref[...]
    lv = jnp.dot(h2, lvW_ref[...], preferred_element_type=jnp.float32) + lvb_ref[...]
    std = jnp.exp(0.5 * lv)
    z_ref[...] = mu + nz_ref[...] * std
    mu_ref[...] = mu
    lv_ref[...] = lv


def _mlp2_heads(h1s, agg2, eps2, m2_W1, m2_b1, m2_W2, m2_b2,
                mu_W, mu_b, lv_W, lv_b, noise_eps):
    grid = N // BN
    wspec = lambda r, c: pl.BlockSpec((r, c), lambda i: (0, 0))
    return pl.pallas_call(
        _mlp2_body,
        grid=(grid,),
        in_specs=[
            pl.BlockSpec((BN, 2 * H), lambda i: (i, 0)),
            pl.BlockSpec((2, BN, 32), lambda i: (0, i, 0)),
            wspec(1, 1), wspec(H, H), wspec(1, H), wspec(H, H), wspec(1, H),
            wspec(H, L), wspec(1, L), wspec(H, L), wspec(1, L),
            pl.BlockSpec((BN, L), lambda i: (i, 0)),
        ],
        out_specs=[pl.BlockSpec((BN, L), lambda i: (i, 0))] * 3,
        out_shape=[jax.ShapeDtypeStruct((N, L), jnp.float32)] * 3,
    )(h1s, agg2, eps2.reshape(1, 1), m2_W1, m2_b1.reshape(1, H), m2_W2,
      m2_b2.reshape(1, H), mu_W, mu_b.reshape(1, L), lv_W, lv_b.reshape(1, L),
      noise_eps)


def _pool_body(z_ref, b_ref, cw_ref, cb_ref, o_ref, sums, counts):
    i = pl.program_id(0)

    @pl.when(i == 0)
    def _():
        sums[...] = jnp.zeros_like(sums)
        counts[...] = jnp.zeros_like(counts)

    iota = lax.broadcasted_iota(jnp.int32, (BN, G), 1)
    onehot = (iota == b_ref[...]).astype(jnp.float32)
    dn = (((0,), (0,)), ((), ()))
    zb = z_ref[...]
    zb = jnp.clip(jnp.where(jnp.isnan(zb), 0.0, zb), -1e38, 1e38)
    sums[...] += lax.dot_general(onehot, zb, dn,
                                 preferred_element_type=jnp.float32)
    counts[...] += lax.dot_general(onehot, jnp.ones((BN, 1), jnp.float32), dn,
                                   preferred_element_type=jnp.float32)

    @pl.when(i == pl.num_programs(0) - 1)
    def _():
        ge = sums[...] / jnp.maximum(counts[...], 1.0)
        o_ref[...] = jnp.dot(ge, cw_ref[...],
                             preferred_element_type=jnp.float32) + cb_ref[...]


def _pool_cls(z, batch2, cls_W, cls_b):
    grid = N // BN
    return pl.pallas_call(
        _pool_body,
        grid=(grid,),
        in_specs=[
            pl.BlockSpec((BN, L), lambda i: (i, 0)),
            pl.BlockSpec((BN, 1), lambda i: (i, 0)),
            pl.BlockSpec((L, C), lambda i: (0, 0)),
            pl.BlockSpec((1, C), lambda i: (0, 0)),
        ],
        out_specs=pl.BlockSpec((G, C), lambda i: (0, 0)),
        out_shape=jax.ShapeDtypeStruct((G, C), jnp.float32),
        scratch_shapes=[
            pltpu.VMEM((G, L), jnp.float32),
            pltpu.VMEM((G, 1), jnp.float32),
        ],
    )(z, batch2, cls_W, cls_b.reshape(1, C))


_SC_MESH = plsc.VectorSubcoreMesh(core_axis_name="c", subcore_axis_name="s")

RW1 = R_PAD // 32
CH1 = 8
NCH1 = RW1 // CH1
ZR1 = N1_PAD // 16


@functools.partial(
    pl.kernel,
    out_type=jax.ShapeDtypeStruct((2 * N1_PAD,), jnp.float32),
    mesh=_SC_MESH,
    scratch_types=[
        pltpu.VMEM((CH1 * 128,), jnp.float32),
        pltpu.VMEM((CH1, 128), jnp.int32),
        pltpu.VMEM((CH1, 128), jnp.int32),
        pltpu.VMEM((CH1 * 128,), jnp.float32),
        pltpu.VMEM((CH1 * 128,), jnp.float32),
        pltpu.VMEM((3136,), jnp.float32),
        pltpu.VMEM_SHARED((N1_PAD,), jnp.float32),
        pltpu.SemaphoreType.DMA,
        pltpu.SemaphoreType.DMA,
    ],
)
def _sc_mp1(x_h, e1_h, srcp, dstp, out, gxb, idxb, dstb, e1b, msgb, zb,
            agg1, sem, sem2):
    c = lax.axis_index("c")
    s = lax.axis_index("s")
    w = s * 2 + c

    def z16(i, t):
        zb[pl.ds(i * 16, 16)] = jnp.zeros((16,), jnp.float32)
        return t

    lax.fori_loop(0, 196, z16, 0, unroll=8)
    z0 = s * ZR1
    for kk in range(3):
        pltpu.sync_copy(zb.at[pl.ds(0, 1024)],
                        agg1.at[pl.ds(z0 + kk * 1024, 1024)])
    pltpu.sync_copy(zb.at[pl.ds(0, 64)], agg1.at[pl.ds(z0 + 3072, 64)])
    plsc.subcore_barrier()

    row0 = w * RW1

    def chunk(k, t):
        st = row0 + k * CH1
        pltpu.sync_copy(srcp.at[pl.ds(st, CH1)], idxb)
        pltpu.sync_copy(dstp.at[pl.ds(st, CH1)], dstb)
        pltpu.sync_copy(e1_h.at[pl.ds(st * 128, CH1 * 128)], e1b)
        gath = [pltpu.async_copy(x_h.at[idxb.at[r]],
                                 gxb.at[pl.ds(r * 128, 128)], sem)
                for r in range(CH1)]
        for g in gath:
            g.wait()

        def body(i, t2):
            sl = pl.ds(i * 16, 16)
            msgb[sl] = jnp.maximum(gxb[sl] + e1b[sl], 0.0)
            return t2

        lax.fori_loop(0, CH1 * 8, body, 0, unroll=8)
        scat = [pltpu.async_copy(msgb.at[pl.ds(r * 128, 128)],
                                 agg1.at[dstb.at[r]], sem2, add=True)
                for r in range(CH1)]
        for t2 in scat:
            t2.wait()
        return t

    lax.fori_loop(0, NCH1, chunk, 0)
    plsc.subcore_barrier()
    o0 = s * ZR1
    pltpu.sync_copy(agg1.at[pl.ds(o0, ZR1)], zb)
    pltpu.sync_copy(zb, out.at[pl.ds(c * N1_PAD + o0, ZR1)])


RW2 = R_PAD // 16
NCH2 = RW2
N2_PAD = 50048
ZR2 = N2_PAD // 16


@functools.partial(
    pl.kernel,
    out_type=jax.ShapeDtypeStruct((2 * N * 32,), jnp.float32),
    mesh=_SC_MESH,
    scratch_types=[
        pltpu.VMEM((1, 128), jnp.int32),
        pltpu.VMEM((1, 128), jnp.int32),
        pltpu.VMEM((128 * 32,), jnp.float32),
        pltpu.VMEM((128, 128), jnp.float32),
        pltpu.VMEM((128, 32), jnp.float32),
        pltpu.VMEM_SHARED((N2_PAD, 32), jnp.float32),
        pltpu.SemaphoreType.DMA,
        pltpu.SemaphoreType.DMA,
    ],
)
def _sc_mp2(h128, e2a, e2b, srcp, dstp, out, idxb, dstb, ebuf, g128,
            msgb, agg, sem, sem2):
    c = lax.axis_index("c")
    s = lax.axis_index("s")

    def zrow(i, t):
        msgb[i, pl.ds(0, 16)] = jnp.zeros((16,), jnp.float32)
        msgb[i, pl.ds(16, 16)] = jnp.zeros((16,), jnp.float32)
        return t

    lax.fori_loop(0, 128, zrow, 0, unroll=8)
    z0 = s * ZR2

    def zcp(kk, t):
        pltpu.sync_copy(msgb, agg.at[pl.ds(z0 + kk * 128, 128)])
        return t

    lax.fori_loop(0, 24, zcp, 0)
    pltpu.sync_copy(msgb.at[pl.ds(0, 56)], agg.at[pl.ds(z0 + 3072, 56)])
    plsc.subcore_barrier()

    row0 = s * RW2

    def chunk(k, t):
        st = row0 + k
        pltpu.sync_copy(srcp.at[pl.ds(st, 1)], idxb)
        pltpu.sync_copy(dstp.at[pl.ds(st, 1)], dstb)

        @pl.when(c == 0)
        def _():
            pltpu.sync_copy(e2a.at[pl.ds(st * 4096, 4096)], ebuf)

        @pl.when(c == 1)
        def _():
            pltpu.sync_copy(e2b.at[pl.ds(st * 4096, 4096)], ebuf)

        pltpu.async_copy(h128.at[idxb.at[0]], g128, sem).wait()

        @pl.when(c == 0)
        def _():
            def body(i, t2):
                a = g128[i, pl.ds(0, 16)] + ebuf[pl.ds(i * 32, 16)]
                msgb[i, pl.ds(0, 16)] = jnp.maximum(a, 0.0)
                b = g128[i, pl.ds(16, 16)] + ebuf[pl.ds(i * 32 + 16, 16)]
                msgb[i, pl.ds(16, 16)] = jnp.maximum(b, 0.0)
                return t2

            lax.fori_loop(0, 128, body, 0, unroll=8)

        @pl.when(c == 1)
        def _():
            def body(i, t2):
                a = g128[i, pl.ds(32, 16)] + ebuf[pl.ds(i * 32, 16)]
                msgb[i, pl.ds(0, 16)] = jnp.maximum(a, 0.0)
                b = g128[i, pl.ds(48, 16)] + ebuf[pl.ds(i * 32 + 16, 16)]
                msgb[i, pl.ds(16, 16)] = jnp.maximum(b, 0.0)
                return t2

            lax.fori_loop(0, 128, body, 0, unroll=8)

        pltpu.async_copy(msgb, agg.at[dstb.at[0]], sem2, add=True).wait()
        return t

    lax.fori_loop(0, NCH2, chunk, 0)
    plsc.subcore_barrier()
    o0 = s * ZR2

    def out_block(base, rows):
        pltpu.sync_copy(agg.at[pl.ds(base, rows)], msgb.at[pl.ds(0, rows)])

        def mv(i, t):
            ebuf[pl.ds(i * 32, 16)] = msgb[i, pl.ds(0, 16)]
            ebuf[pl.ds(i * 32 + 16, 16)] = msgb[i, pl.ds(16, 16)]
            return t

        lax.fori_loop(0, rows, mv, 0, unroll=8)
        pltpu.sync_copy(ebuf.at[pl.ds(0, rows * 32)],
                        out.at[pl.ds((c * N + base) * 32, rows * 32)])

    def ocp(kk, t):
        out_block(o0 + kk * 64, 64)
        return t

    lax.fori_loop(0, 48, ocp, 0)

    @pl.when(s < 15)
    def _():
        out_block(o0 + 3072, 56)

    @pl.when(s == 15)
    def _():
        out_block(o0 + 3072, 8)


def kernel(x, edge_index, edge_attr, batch,
           eps1, ep1_W, ep1_b, m1_W1, m1_b1, m1_W2, m1_b2,
           eps2, ep2_W, ep2_b, m2_W1, m2_b1, m2_W2, m2_b2,
           mu_W, mu_b, lv_W, lv_b, cls_W, cls_b, noise_eps):
    src = edge_index[0].astype(jnp.int32)
    dst = edge_index[1].astype(jnp.int32)

    e1, e2a, e2b = _edge_proj(edge_attr, ep1_W, ep1_b, ep2_W, ep2_b)

    pad = E_PAD - E
    srcp = jnp.concatenate([src, jnp.zeros((pad,), jnp.int32)]).reshape(
        R_PAD, 128)
    dstp = jnp.concatenate([dst, jnp.full((pad,), TRASH, jnp.int32)]).reshape(
        R_PAD, 128)

    agg1p = _sc_mp1(x, e1.reshape(E_PAD), srcp, dstp)
    h1p = _mlp1(x.reshape(N, 1), agg1p.reshape(2, N1_PAD, 1), eps1,
                m1_W1, m1_b1, m1_W2, m1_b2)

    e2e = jnp.concatenate([e2a[:E], e2b[:E]], axis=1)
    msg2 = jax.nn.relu(jnp.take(h1p[:, :H], src, axis=0) + e2e)
    agg2f = jax.ops.segment_sum(msg2, dst, num_segments=N)
    agg2 = jnp.stack([agg2f[:, :32], agg2f[:, 32:]])

    z, mu, lv = _mlp2_heads(h1p, agg2, eps2, m2_W1, m2_b1, m2_W2, m2_b2,
                            mu_W, mu_b, lv_W, lv_b, noise_eps)

    logits = _pool_cls(z, batch.reshape(N, 1), cls_W, cls_b)
    return (z, mu, lv, logits)
